# baseline (speedup 1.0000x reference)
import functools

import functools

import jax
import jax.numpy as jnp
from jax import lax
from jax.experimental import pallas as pl
from jax.experimental.pallas import tpu as pltpu
from jax.experimental.pallas import tpu_sc as plsc

NC, NS = 2, 16
NW = NC * NS
CHUNK = 25000
NBUF = 4

_sc_mesh = plsc.VectorSubcoreMesh(core_axis_name="c", subcore_axis_name="s")


def _make_sc_stream(total_elems):
    spw = total_elems // NW
    nchunk = spw // CHUNK

    @functools.partial(
        pl.kernel,
        mesh=_sc_mesh,
        out_type=jax.ShapeDtypeStruct((NW, 16), jnp.float32),
        scratch_types=(
            [pltpu.VMEM((CHUNK,), jnp.float32) for _ in range(NBUF)]
            + [pltpu.VMEM((16,), jnp.float32)]
            + [pltpu.SemaphoreType.DMA for _ in range(NBUF)]
        ),
    )
    def sc_stream(a_hbm, out_hbm, *refs):
        bufs = refs[:NBUF]
        acc_v = refs[NBUF]
        sems = refs[NBUF + 1:]
        cid = lax.axis_index("c")
        sid = lax.axis_index("s")
        wid = sid * NC + cid
        base = wid * spw

        def start(c, b):
            off = base + jnp.minimum(c, nchunk - 1) * CHUNK
            pltpu.make_async_copy(
                a_hbm.at[pl.ds(off, CHUNK)], bufs[b], sems[b]).start()

        def wait(c, b):
            off = base + jnp.minimum(c, nchunk - 1) * CHUNK
            pltpu.make_async_copy(
                a_hbm.at[pl.ds(off, CHUNK)], bufs[b], sems[b]).wait()

        for b in range(NBUF):
            start(b, b)

        def chunk_body(c, carry):
            c0 = NBUF * c
            for b in range(NBUF):
                wait(c0 + b, b)
                start(c0 + b + NBUF, b)
            return carry

        carry = lax.fori_loop(0, nchunk // NBUF, chunk_body, 0)
        for b in range(NBUF):
            wait(nchunk, b)

        acc_v[...] = jnp.zeros((16,), jnp.float32) + carry
        pltpu.sync_copy(acc_v, out_hbm.at[wid])

    return sc_stream


TAU = 0.8


def _proj_kernel(x_ref, w_ref, b_ref, out_ref):
    z = jax.lax.dot_general(
        x_ref[...], w_ref[...],
        dimension_numbers=(((1,), (1,)), ((), ())),
        preferred_element_type=jnp.float32,
    ) + b_ref[...]
    z = jnp.where(z > 0, z, jnp.exp(z) - 1.0)
    norm = jnp.sqrt(jnp.sum(z * z, axis=1, keepdims=True))
    out_ref[...] = z / norm


def _sim_kernel(zn1_ref, zn2_ref, pos_ref, neg_ref, out_p_ref, out_n_ref):
    s = jax.lax.dot_general(
        zn1_ref[...], zn2_ref[...],
        dimension_numbers=(((1,), (1,)), ((), ())),
        preferred_element_type=jnp.float32,
    )
    s = jnp.exp(s * (1.0 / TAU))
    out_p_ref[0, 0, 0] = jnp.sum(s * pos_ref[...])
    out_n_ref[0, 0, 0] = jnp.sum(s * neg_ref[...])


def _largest_divisor(n, cap):
    for c in range(min(cap, n), 0, -1):
        if n % c == 0:
            return c
    return n


@jax.jit
def kernel(v1_embs, v2_embs, pos, neg, W, b):
    n, d = v1_embs.shape
    sc_stream = _make_sc_stream(41600000)
    sc_side = sc_stream(pos.reshape(n * n))

    br = _largest_divisor(n, 2000)
    proj = pl.pallas_call(
        _proj_kernel,
        grid=(n // br,),
        in_specs=[
            pl.BlockSpec((br, d), lambda i: (i, 0)),
            pl.BlockSpec((d, d), lambda i: (0, 0)),
            pl.BlockSpec((1, d), lambda i: (0, 0)),
        ],
        out_specs=pl.BlockSpec((br, d), lambda i: (i, 0)),
        out_shape=jax.ShapeDtypeStruct((n, d), jnp.float32),
    )
    b2 = b.reshape(1, d)
    zn1 = proj(v1_embs, W, b2)
    zn2 = proj(v2_embs, W, b2)

    bm = _largest_divisor(n, 200) if n % 8 == 0 else n
    ni = n // bm
    part_p, part_n = pl.pallas_call(
        _sim_kernel,
        grid=(ni,),
        in_specs=[
            pl.BlockSpec((bm, d), lambda i: (i, 0)),
            pl.BlockSpec((n, d), lambda i: (0, 0)),
            pl.BlockSpec((bm, n), lambda i: (i, 0)),
            pl.BlockSpec((bm, n), lambda i: (i, 0)),
        ],
        out_specs=[
            pl.BlockSpec((1, 1, 1), lambda i: (i, 0, 0), memory_space=pltpu.SMEM),
            pl.BlockSpec((1, 1, 1), lambda i: (i, 0, 0), memory_space=pltpu.SMEM),
        ],
        out_shape=[
            jax.ShapeDtypeStruct((ni, 1, 1), jnp.float32),
            jax.ShapeDtypeStruct((ni, 1, 1), jnp.float32),
        ],
        compiler_params=pltpu.CompilerParams(
            dimension_semantics=("parallel",),
        ),
    )(zn1, zn2, pos, neg)

    sum_p = jnp.sum(part_p) + jnp.sum(sc_side) * 1e-38
    sum_n = jnp.sum(part_n)
    return jnp.log(sum_p + sum_n) - jnp.log(sum_p)


# speedup vs baseline: 2.6695x; 2.6695x over previous
import jax
import jax.numpy as jnp
from jax.experimental import pallas as pl
from jax.experimental.pallas import tpu as pltpu

TAU = 0.8


def _proj_normalize(x, w, b):
    z = jax.lax.dot_general(
        x, w,
        dimension_numbers=(((1,), (1,)), ((), ())),
        preferred_element_type=jnp.float32,
    ) + b
    z = jnp.where(z > 0, z, jnp.exp(z) - 1.0)
    norm = jnp.sqrt(jnp.sum(z * z, axis=1, keepdims=True))
    return z / norm


def _fused_kernel(v1_ref, v2_ref, w_ref, b_ref, pos_ref, neg_ref,
                  loss_ref, zn2_ref, acc_p, acc_n):
    i = pl.program_id(0)

    @pl.when(i == 0)
    def _():
        zn2_ref[...] = _proj_normalize(v2_ref[...], w_ref[...], b_ref[...])
        acc_p[0, 0] = 0.0
        acc_n[0, 0] = 0.0

    zn1 = _proj_normalize(v1_ref[...], w_ref[...], b_ref[...])
    s = jax.lax.dot_general(
        zn1, zn2_ref[...],
        dimension_numbers=(((1,), (1,)), ((), ())),
        preferred_element_type=jnp.float32,
    )
    s = jnp.exp(s * (1.0 / TAU))
    acc_p[0, 0] += jnp.sum(s * pos_ref[...])
    acc_n[0, 0] += jnp.sum(s * neg_ref[...])

    @pl.when(i == pl.num_programs(0) - 1)
    def _():
        sum_p = acc_p[0, 0]
        sum_n = acc_n[0, 0]
        loss_ref[0, 0] = jnp.log(sum_p + sum_n) - jnp.log(sum_p)


def _largest_divisor(n, cap):
    for c in range(min(cap, n), 0, -1):
        if n % c == 0:
            return c
    return n


@jax.jit
def kernel(v1_embs, v2_embs, pos, neg, W, b):
    n, d = v1_embs.shape
    bm = _largest_divisor(n, 200) if n % 8 == 0 else n
    ni = n // bm
    loss = pl.pallas_call(
        _fused_kernel,
        grid=(ni,),
        in_specs=[
            pl.BlockSpec((bm, d), lambda i: (i, 0)),
            pl.BlockSpec((n, d), lambda i: (0, 0)),
            pl.BlockSpec((d, d), lambda i: (0, 0)),
            pl.BlockSpec((1, d), lambda i: (0, 0)),
            pl.BlockSpec((bm, n), lambda i: (i, 0)),
            pl.BlockSpec((bm, n), lambda i: (i, 0)),
        ],
        out_specs=pl.BlockSpec((1, 1), lambda i: (0, 0),
                               memory_space=pltpu.SMEM),
        out_shape=jax.ShapeDtypeStruct((1, 1), jnp.float32),
        scratch_shapes=[
            pltpu.VMEM((n, d), jnp.float32),
            pltpu.SMEM((1, 1), jnp.float32),
            pltpu.SMEM((1, 1), jnp.float32),
        ],
        compiler_params=pltpu.CompilerParams(
            dimension_semantics=("arbitrary",),
        ),
    )(v1_embs, v2_embs, W, b.reshape(1, d), pos, neg)
    return loss[0, 0]
